# initial kernel scaffold (unmeasured)
import jax
import jax.numpy as jnp
from jax import lax
from jax.experimental import pallas as pl
from jax.experimental.pallas import tpu as pltpu

N_DEV = 8


def kernel(x, W1, W2):
    m, k = x.shape
    n = W2.shape[1]
    chunk = m // N_DEV

    def body(x_ref, w1_ref, w2_ref, out_ref, rs_recv,
             rs_send_sems, rs_recv_sems, ag_send_sems, ag_recv_sems):
        me = lax.axis_index("i")
        left = lax.rem(me - 1 + N_DEV, N_DEV)
        right = lax.rem(me + 1, N_DEV)

        barrier_sem = pltpu.get_barrier_semaphore()
        for nbr in (left, right):
            pl.semaphore_signal(
                barrier_sem, inc=1,
                device_id=(nbr,), device_id_type=pl.DeviceIdType.MESH,
            )
        pl.semaphore_wait(barrier_sem, 2)

        h = jnp.maximum(
            jnp.dot(x_ref[...], w1_ref[...], preferred_element_type=jnp.float32),
            0.0,
        )
        out_ref[...] = jnp.dot(h, w2_ref[...], preferred_element_type=jnp.float32)

        for s in range(N_DEV - 1):
            c_send = lax.rem(me - s + N_DEV, N_DEV)
            rdma = pltpu.make_async_remote_copy(
                src_ref=out_ref.at[pl.ds(c_send * chunk, chunk), :],
                dst_ref=rs_recv.at[s],
                send_sem=rs_send_sems.at[s],
                recv_sem=rs_recv_sems.at[s],
                device_id=(right,),
                device_id_type=pl.DeviceIdType.MESH,
            )
            rdma.start()
            rdma.wait()
            c_recv = lax.rem(me - s - 1 + N_DEV, N_DEV)
            sl = pl.ds(c_recv * chunk, chunk)
            out_ref[sl, :] = out_ref[sl, :] + rs_recv[s]

        for s in range(N_DEV - 1):
            c_send = lax.rem(me + 1 - s + 2 * N_DEV, N_DEV)
            sl_send = pl.ds(c_send * chunk, chunk)
            rdma = pltpu.make_async_remote_copy(
                src_ref=out_ref.at[sl_send, :],
                dst_ref=out_ref.at[sl_send, :],
                send_sem=ag_send_sems.at[s],
                recv_sem=ag_recv_sems.at[s],
                device_id=(right,),
                device_id_type=pl.DeviceIdType.MESH,
            )
            rdma.start()
            rdma.wait()

    return pl.pallas_call(
        body,
        out_shape=jax.ShapeDtypeStruct((m, n), jnp.float32),
        in_specs=[
            pl.BlockSpec(memory_space=pltpu.VMEM),
            pl.BlockSpec(memory_space=pltpu.VMEM),
            pl.BlockSpec(memory_space=pltpu.VMEM),
        ],
        out_specs=pl.BlockSpec(memory_space=pltpu.VMEM),
        scratch_shapes=[
            pltpu.VMEM((N_DEV - 1, chunk, n), jnp.float32),
            pltpu.SemaphoreType.DMA((N_DEV - 1,)),
            pltpu.SemaphoreType.DMA((N_DEV - 1,)),
            pltpu.SemaphoreType.DMA((N_DEV - 1,)),
            pltpu.SemaphoreType.DMA((N_DEV - 1,)),
        ],
        compiler_params=pltpu.CompilerParams(collective_id=0),
    )(x, W1, W2)


# baseline (device time: 253905 ns/iter reference)
import jax
import jax.numpy as jnp
from jax import lax
from jax.experimental import pallas as pl
from jax.experimental.pallas import tpu as pltpu

N_DEV = 8
HB = 512


def kernel(x, W1, W2):
    m, k = x.shape
    h_per = W1.shape[1]
    n = W2.shape[1]
    chunk = m // N_DEV
    grid = h_per // HB

    def body(x_ref, w1_ref, w2_ref, out_ref, rs_recv,
             rs_send_sems, rs_recv_sems, ag_send_sems, ag_recv_sems):
        j = pl.program_id(0)
        me = lax.axis_index("i")
        left = lax.rem(me - 1 + N_DEV, N_DEV)
        right = lax.rem(me + 1, N_DEV)

        @pl.when(j == 0)
        def _():
            barrier_sem = pltpu.get_barrier_semaphore()
            for nbr in (left, right):
                pl.semaphore_signal(
                    barrier_sem, inc=1,
                    device_id=(nbr,), device_id_type=pl.DeviceIdType.MESH,
                )
            pl.semaphore_wait(barrier_sem, 2)

        h = jnp.maximum(
            jnp.dot(x_ref[...], w1_ref[...], preferred_element_type=jnp.float32),
            0.0,
        )
        p = jnp.dot(h, w2_ref[...], preferred_element_type=jnp.float32)

        @pl.when(j == 0)
        def _():
            out_ref[...] = p

        @pl.when(j > 0)
        def _():
            out_ref[...] = out_ref[...] + p

        @pl.when(j == grid - 1)
        def _():
            for s in range(N_DEV - 1):
                c_send = lax.rem(me - s + N_DEV, N_DEV)
                rdma = pltpu.make_async_remote_copy(
                    src_ref=out_ref.at[pl.ds(c_send * chunk, chunk), :],
                    dst_ref=rs_recv.at[s],
                    send_sem=rs_send_sems.at[s],
                    recv_sem=rs_recv_sems.at[s],
                    device_id=(right,),
                    device_id_type=pl.DeviceIdType.MESH,
                )
                rdma.start()
                rdma.wait()
                c_recv = lax.rem(me - s - 1 + N_DEV, N_DEV)
                sl = pl.ds(c_recv * chunk, chunk)
                out_ref[sl, :] = out_ref[sl, :] + rs_recv[s]

            for s in range(N_DEV - 1):
                c_send = lax.rem(me + 1 - s + 2 * N_DEV, N_DEV)
                sl_send = pl.ds(c_send * chunk, chunk)
                rdma = pltpu.make_async_remote_copy(
                    src_ref=out_ref.at[sl_send, :],
                    dst_ref=out_ref.at[sl_send, :],
                    send_sem=ag_send_sems.at[s],
                    recv_sem=ag_recv_sems.at[s],
                    device_id=(right,),
                    device_id_type=pl.DeviceIdType.MESH,
                )
                rdma.start()
                rdma.wait()

    return pl.pallas_call(
        body,
        grid=(grid,),
        out_shape=jax.ShapeDtypeStruct((m, n), jnp.float32),
        in_specs=[
            pl.BlockSpec((m, k), lambda j: (0, 0)),
            pl.BlockSpec((k, HB), lambda j: (0, j)),
            pl.BlockSpec((HB, n), lambda j: (j, 0)),
        ],
        out_specs=pl.BlockSpec((m, n), lambda j: (0, 0)),
        scratch_shapes=[
            pltpu.VMEM((N_DEV - 1, chunk, n), jnp.float32),
            pltpu.SemaphoreType.DMA((N_DEV - 1,)),
            pltpu.SemaphoreType.DMA((N_DEV - 1,)),
            pltpu.SemaphoreType.DMA((N_DEV - 1,)),
            pltpu.SemaphoreType.DMA((N_DEV - 1,)),
        ],
        compiler_params=pltpu.CompilerParams(collective_id=0),
    )(x, W1, W2)


# device time: 100456 ns/iter; 2.5275x vs baseline; 2.5275x over previous
import jax
import jax.numpy as jnp
from jax import lax
from jax.experimental import pallas as pl
from jax.experimental.pallas import tpu as pltpu

N_DEV = 8
HB = 512
NS = 3
SLAB = 512

MASKS = {"x": 1, "y": 3, "z": 4}
DIM_ORDERS = [("x", "y", "z"), ("y", "z", "x"), ("z", "x", "y")]
RS_OFF = (0, 768, 1152)
RS_LEN = (768, 384, 192)


def kernel(x, W1, W2):
    m, k = x.shape
    h_per = W1.shape[1]
    n = W2.shape[1]
    grid = h_per // HB

    def body(x_ref, w1_ref, w2_ref, out_ref,
             rs_stage, rs_recv, gat,
             rs_send_sems, rs_recv_sems, ag_send_sems, ag_recv_sems):
        j = pl.program_id(0)
        me = lax.axis_index("i")

        bit0 = lax.rem(me, 2)
        bit1 = lax.rem(lax.div(me, 2), 2)
        bit2 = lax.div(me, 4)
        side = {"x": lax.rem(bit0 + bit1, 2), "y": bit1, "z": bit2}
        partner = {d: jnp.bitwise_xor(me, mk) for d, mk in MASKS.items()}

        @pl.when(j == 0)
        def _():
            barrier_sem = pltpu.get_barrier_semaphore()
            for d in ("x", "y", "z"):
                pl.semaphore_signal(
                    barrier_sem, inc=1,
                    device_id=(partner[d],),
                    device_id_type=pl.DeviceIdType.MESH,
                )
            pl.semaphore_wait(barrier_sem, 3)

        h = jnp.maximum(
            jnp.dot(x_ref[...], w1_ref[...], preferred_element_type=jnp.float32),
            0.0,
        )
        p = jnp.dot(h, w2_ref[...], preferred_element_type=jnp.float32)

        @pl.when(j == 0)
        def _():
            out_ref[...] = p

        @pl.when(j > 0)
        def _():
            out_ref[...] = out_ref[...] + p

        @pl.when(j == grid - 1)
        def _():
            csl = [pl.ds(s * SLAB, SLAB) for s in range(NS)]

            cur_start = [jnp.int32(0) for _ in range(NS)]
            for kk in range(3):
                half = 1536 >> (kk + 1)
                send_start = []
                for s in range(NS):
                    d = DIM_ORDERS[s][kk]
                    b = side[d]
                    snd = cur_start[s] + (1 - b) * half
                    send_start.append(snd)
                    rs_stage[s, pl.ds(RS_OFF[kk], half), :] = (
                        out_ref[pl.ds(snd, half), csl[s]].astype(jnp.bfloat16)
                    )
                rdmas = []
                for s in range(NS):
                    d = DIM_ORDERS[s][kk]
                    rdma = pltpu.make_async_remote_copy(
                        src_ref=rs_stage.at[s, pl.ds(RS_OFF[kk], half), :],
                        dst_ref=rs_recv.at[s, pl.ds(RS_OFF[kk], half), :],
                        send_sem=rs_send_sems.at[s, kk],
                        recv_sem=rs_recv_sems.at[s, kk],
                        device_id=(partner[d],),
                        device_id_type=pl.DeviceIdType.MESH,
                    )
                    rdma.start()
                    rdmas.append(rdma)
                for s in range(NS):
                    d = DIM_ORDERS[s][kk]
                    b = side[d]
                    rdmas[s].wait()
                    keep = cur_start[s] + b * half
                    sl = pl.ds(keep, half)
                    out_ref[sl, csl[s]] = (
                        out_ref[sl, csl[s]]
                        + rs_recv[s, pl.ds(RS_OFF[kk], half), :].astype(jnp.float32)
                    )
                    cur_start[s] = keep

            for s in range(NS):
                gat[s, pl.ds(cur_start[s], 192), :] = (
                    out_ref[pl.ds(cur_start[s], 192), csl[s]].astype(jnp.bfloat16)
                )
            have_start = list(cur_start)
            for tt in range(3):
                have = 192 << tt
                kk = 2 - tt
                rdmas = []
                for s in range(NS):
                    d = DIM_ORDERS[s][kk]
                    rdma = pltpu.make_async_remote_copy(
                        src_ref=gat.at[s, pl.ds(have_start[s], have), :],
                        dst_ref=gat.at[s, pl.ds(have_start[s], have), :],
                        send_sem=ag_send_sems.at[s, tt],
                        recv_sem=ag_recv_sems.at[s, tt],
                        device_id=(partner[d],),
                        device_id_type=pl.DeviceIdType.MESH,
                    )
                    rdma.start()
                    rdmas.append(rdma)
                for s in range(NS):
                    d = DIM_ORDERS[s][kk]
                    b = side[d]
                    rdmas[s].wait()
                    union = have_start[s] - b * have
                    pstart = union + (1 - b) * have
                    out_ref[pl.ds(pstart, have), csl[s]] = (
                        gat[s, pl.ds(pstart, have), :].astype(jnp.float32)
                    )
                    have_start[s] = union

    return pl.pallas_call(
        body,
        grid=(grid,),
        out_shape=jax.ShapeDtypeStruct((m, n), jnp.float32),
        in_specs=[
            pl.BlockSpec((m, k), lambda j: (0, 0)),
            pl.BlockSpec((k, HB), lambda j: (0, j)),
            pl.BlockSpec((HB, n), lambda j: (j, 0)),
        ],
        out_specs=pl.BlockSpec((m, n), lambda j: (0, 0)),
        scratch_shapes=[
            pltpu.VMEM((NS, 1344, SLAB), jnp.bfloat16),
            pltpu.VMEM((NS, 1344, SLAB), jnp.bfloat16),
            pltpu.VMEM((NS, 1536, SLAB), jnp.bfloat16),
            pltpu.SemaphoreType.DMA((NS, 3)),
            pltpu.SemaphoreType.DMA((NS, 3)),
            pltpu.SemaphoreType.DMA((NS, 3)),
            pltpu.SemaphoreType.DMA((NS, 3)),
        ],
        compiler_params=pltpu.CompilerParams(
            collective_id=0, vmem_limit_bytes=60 * 1024 * 1024
        ),
    )(x, W1, W2)


# device time: 95529 ns/iter; 2.6579x vs baseline; 1.0516x over previous
import jax
import jax.numpy as jnp
from jax import lax
from jax.experimental import pallas as pl
from jax.experimental.pallas import tpu as pltpu

N_DEV = 8
HB = 512
NS = 3
SLAB = 512

MASKS = {"x": 1, "y": 3, "z": 4}
DIM_ORDERS = [("x", "y", "z"), ("y", "z", "x"), ("z", "x", "y")]


def kernel(x, W1, W2):
    m, k = x.shape
    h_per = W1.shape[1]
    n = W2.shape[1]
    grid = h_per // HB

    def body(x_ref, w1_ref, w2_ref, out_ref,
             rs_stage, rs_recv, gat,
             rs_send_sems, rs_recv_sems, ag_send_sems, ag_recv_sems):
        j = pl.program_id(0)
        me = lax.axis_index("i")

        bit0 = lax.rem(me, 2)
        bit1 = lax.rem(lax.div(me, 2), 2)
        bit2 = lax.div(me, 4)
        side = {"x": lax.rem(bit0 + bit1, 2), "y": bit1, "z": bit2}
        partner = {d: jnp.bitwise_xor(me, mk) for d, mk in MASKS.items()}

        @pl.when(j == 0)
        def _():
            barrier_sem = pltpu.get_barrier_semaphore()
            for d in ("x", "y", "z"):
                pl.semaphore_signal(
                    barrier_sem, inc=1,
                    device_id=(partner[d],),
                    device_id_type=pl.DeviceIdType.MESH,
                )
            pl.semaphore_wait(barrier_sem, 3)

        h = jnp.maximum(
            jnp.dot(x_ref[...], w1_ref[...], preferred_element_type=jnp.float32),
            0.0,
        )
        p = jnp.dot(h, w2_ref[...], preferred_element_type=jnp.float32)

        @pl.when(j == 0)
        def _():
            out_ref[...] = p

        @pl.when(jnp.logical_and(j > 0, j < grid - 1))
        def _():
            out_ref[...] = out_ref[...] + p

        @pl.when(j == grid - 1)
        def _():
            csl = [pl.ds(s * SLAB, SLAB) for s in range(NS)]
            inflight = {}

            def rs_xfer(s, idx, dim, start, length, dst_start=None):
                dst = start if dst_start is None else dst_start
                rdma = pltpu.make_async_remote_copy(
                    src_ref=rs_stage.at[s, pl.ds(start, length), :],
                    dst_ref=rs_recv.at[s, pl.ds(dst, length), :],
                    send_sem=rs_send_sems.at[s, idx],
                    recv_sem=rs_recv_sems.at[s, idx],
                    device_id=(partner[dim],),
                    device_id_type=pl.DeviceIdType.MESH,
                )
                rdma.start()
                inflight[(s, "rs", idx)] = rdma

            def ag_xfer(s, idx, dim, start, length):
                rdma = pltpu.make_async_remote_copy(
                    src_ref=gat.at[s, pl.ds(start, length), :],
                    dst_ref=gat.at[s, pl.ds(start, length), :],
                    send_sem=ag_send_sems.at[s, idx],
                    recv_sem=ag_recv_sems.at[s, idx],
                    device_id=(partner[dim],),
                    device_id_type=pl.DeviceIdType.MESH,
                )
                rdma.start()
                inflight[(s, "ag", idx)] = rdma

            b0 = [side[DIM_ORDERS[s][0]] for s in range(NS)]
            b1 = [side[DIM_ORDERS[s][1]] for s in range(NS)]
            b2 = [side[DIM_ORDERS[s][2]] for s in range(NS)]
            d0 = [DIM_ORDERS[s][0] for s in range(NS)]
            d1 = [DIM_ORDERS[s][1] for s in range(NS)]
            d2 = [DIM_ORDERS[s][2] for s in range(NS)]
            s0 = [(1 - b0[s]) * 768 for s in range(NS)]
            k0 = [b0[s] * 768 for s in range(NS)]
            q1 = [k0[s] + (1 - b1[s]) * 384 for s in range(NS)]
            k1 = [k0[s] + b1[s] * 384 for s in range(NS)]
            q2 = [k1[s] + (1 - b2[s]) * 192 for s in range(NS)]
            o_ = [k1[s] + b2[s] * 192 for s in range(NS)]
            p0 = [k0[s] + b1[s] * 384 + (1 - b2[s]) * 192
                  for s in range(NS)]
            a1r1 = [k0[s] + (1 - b1[s]) * 384 + b2[s] * 192
                    for s in range(NS)]
            a1r2 = [k0[s] + (1 - b1[s]) * 384 + (1 - b2[s]) * 192
                    for s in range(NS)]
            u0 = [k0[s] + b1[s] * 384 for s in range(NS)]
            u1b = [k0[s] + (1 - b1[s]) * 384 for s in range(NS)]
            a2r1 = [(1 - b0[s]) * 768 + b1[s] * 384 for s in range(NS)]
            a2r2 = [(1 - b0[s]) * 768 + (1 - b1[s]) * 384 for s in range(NS)]

            for s in range(NS):
                acc = out_ref[:, csl[s]] + p[:, s * SLAB:(s + 1) * SLAB]
                out_ref[:, csl[s]] = acc
                sub1 = s0[s] + (1 - b1[s]) * 384
                rs_stage[s, pl.ds(sub1, 384), :] = (
                    out_ref[pl.ds(sub1, 384), csl[s]].astype(jnp.bfloat16)
                )
                rs_xfer(s, 0, d0[s], sub1, 384)
                sub2 = s0[s] + b1[s] * 384
                rs_stage[s, pl.ds(sub2, 384), :] = (
                    out_ref[pl.ds(sub2, 384), csl[s]].astype(jnp.bfloat16)
                )
                rs_xfer(s, 1, d0[s], sub2, 384)

            for s in range(NS):
                inflight[(s, "rs", 0)].wait_recv()
                tmp = (
                    out_ref[pl.ds(q1[s], 384), csl[s]]
                    + rs_recv[s, pl.ds(q1[s], 384), :].astype(jnp.float32)
                )
                out_ref[pl.ds(q1[s], 384), csl[s]] = tmp
                rs_stage[s, pl.ds(q1[s], 384), :] = tmp.astype(jnp.bfloat16)
                rs_xfer(s, 2, d1[s], q1[s], 384, dst_start=s0[s])

            for s in range(NS):
                inflight[(s, "rs", 1)].wait_recv()
                out_ref[pl.ds(k1[s], 384), csl[s]] = (
                    out_ref[pl.ds(k1[s], 384), csl[s]]
                    + rs_recv[s, pl.ds(k1[s], 384), :].astype(jnp.float32)
                )

            for s in range(NS):
                inflight[(s, "rs", 2)].wait_recv()
                q2_rel = s0[s] + (1 - b2[s]) * 192
                o_rel = s0[s] + b2[s] * 192
                tmp = (
                    out_ref[pl.ds(q2[s], 192), csl[s]]
                    + rs_recv[s, pl.ds(q2_rel, 192), :].astype(jnp.float32)
                )
                out_ref[pl.ds(q2[s], 192), csl[s]] = tmp
                rs_stage[s, pl.ds(q2[s], 192), :] = tmp.astype(jnp.bfloat16)
                rs_xfer(s, 3, d2[s], q2[s], 192, dst_start=s0[s] + 384)
                out_ref[pl.ds(o_[s], 192), csl[s]] = (
                    out_ref[pl.ds(o_[s], 192), csl[s]]
                    + rs_recv[s, pl.ds(o_rel, 192), :].astype(jnp.float32)
                )

            for s in range(NS):
                inflight[(s, "rs", 3)].wait_recv()
                tmp = (
                    out_ref[pl.ds(o_[s], 192), csl[s]]
                    + rs_recv[s, pl.ds(s0[s] + 384, 192), :].astype(jnp.float32)
                )
                out_ref[pl.ds(o_[s], 192), csl[s]] = tmp
                gat[s, pl.ds(o_[s], 192), :] = tmp.astype(jnp.bfloat16)
                ag_xfer(s, 0, d2[s], o_[s], 192)
                ag_xfer(s, 1, d1[s], o_[s], 192)

            for s in range(NS):
                inflight[(s, "ag", 0)].wait_recv()
                ag_xfer(s, 2, d1[s], p0[s], 192)
                ag_xfer(s, 3, d0[s], u0[s], 384)
                out_ref[pl.ds(p0[s], 192), csl[s]] = (
                    gat[s, pl.ds(p0[s], 192), :].astype(jnp.float32)
                )

            for s in range(NS):
                inflight[(s, "ag", 1)].wait_recv()
                inflight[(s, "ag", 2)].wait_recv()
                ag_xfer(s, 4, d0[s], u1b[s], 384)
                out_ref[pl.ds(u1b[s], 384), csl[s]] = (
                    gat[s, pl.ds(u1b[s], 384), :].astype(jnp.float32)
                )

            for s in range(NS):
                inflight[(s, "ag", 3)].wait_recv()
                out_ref[pl.ds(a2r1[s], 384), csl[s]] = (
                    gat[s, pl.ds(a2r1[s], 384), :].astype(jnp.float32)
                )
            for s in range(NS):
                inflight[(s, "ag", 4)].wait_recv()
                out_ref[pl.ds(a2r2[s], 384), csl[s]] = (
                    gat[s, pl.ds(a2r2[s], 384), :].astype(jnp.float32)
                )

            for key in sorted(inflight):
                inflight[key].wait_send()

    return pl.pallas_call(
        body,
        grid=(grid,),
        out_shape=jax.ShapeDtypeStruct((m, n), jnp.float32),
        in_specs=[
            pl.BlockSpec((m, k), lambda j: (0, 0)),
            pl.BlockSpec((k, HB), lambda j: (0, j)),
            pl.BlockSpec((HB, n), lambda j: (j, 0)),
        ],
        out_specs=pl.BlockSpec((m, n), lambda j: (0, 0)),
        scratch_shapes=[
            pltpu.VMEM((NS, 1536, SLAB), jnp.bfloat16),
            pltpu.VMEM((NS, 1536, SLAB), jnp.bfloat16),
            pltpu.VMEM((NS, 1536, SLAB), jnp.bfloat16),
            pltpu.SemaphoreType.DMA((NS, 4)),
            pltpu.SemaphoreType.DMA((NS, 4)),
            pltpu.SemaphoreType.DMA((NS, 5)),
            pltpu.SemaphoreType.DMA((NS, 5)),
        ],
        compiler_params=pltpu.CompilerParams(
            collective_id=0, vmem_limit_bytes=63 * 1024 * 1024
        ),
    )(x, W1, W2)
